# baseline (speedup 1.0000x reference)
import jax
import jax.numpy as jnp
from jax import lax
from jax.experimental import pallas as pl
from jax.experimental.pallas import tpu as pltpu
from jax.experimental.pallas import tpu_sc as plsc

N_NODES = 512
N_EDGES = 262144
H = 32
DENSE = 128
NC = 2
NS = 16
NW = NC * NS
EPT = N_EDGES // NW
CHUNK = 128
NCHUNK = EPT // CHUNK
SEG = N_EDGES // NS
ZB = 2048
BD = 64
NG = BD // 4


def _sc_mask_body(e_hbm, out_hbm, src_v, dst_v, idx_v, ones_v, zeros_v,
                  mask_sh, sem):
    cid = lax.axis_index("c")
    sid = lax.axis_index("s")
    wid = cid * NS + sid
    base = wid * EPT

    pltpu.sync_copy(e_hbm.at[0, pl.ds(base, EPT)], src_v)
    pltpu.sync_copy(e_hbm.at[1, pl.ds(base, EPT)], dst_v)

    for k in range(CHUNK // 16):
        ones_v[pl.ds(k * 16, 16)] = jnp.ones((16,), jnp.float32)

    def _fill_zeros(i, c):
        for k in range(8):
            zeros_v[pl.ds((i * 8 + k) * 16, 16)] = jnp.zeros((16,),
                                                             jnp.float32)
        return c

    lax.fori_loop(0, ZB // 128, _fill_zeros, 0)

    for k in range(SEG // ZB):
        pltpu.sync_copy(zeros_v, mask_sh.at[pl.ds(sid * SEG + k * ZB, ZB)])

    def _codes(j, c):
        for k in range(8):
            off = j * CHUNK + k * 16
            s = src_v[pl.ds(off, 16)]
            d = dst_v[pl.ds(off, 16)]
            idx_v[j, pl.ds(k * 16, 16)] = d * N_NODES + s
        return c

    lax.fori_loop(0, NCHUNK, _codes, 0)
    plsc.subcore_barrier()

    def _scatter(g, c):
        sds = [pltpu.async_copy(ones_v, mask_sh.at[idx_v.at[g * 8 + k]], sem,
                                add=True)
               for k in range(8)]
        for sd in sds:
            sd.wait()
        return c

    lax.fori_loop(0, NCHUNK // 8, _scatter, 0)
    plsc.subcore_barrier()
    pltpu.sync_copy(mask_sh.at[pl.ds(sid * SEG, SEG)],
                    out_hbm.at[cid, pl.ds(sid * SEG, SEG)])


def _sc_mask(edge_index):
    mesh = plsc.VectorSubcoreMesh(core_axis_name="c", subcore_axis_name="s")
    return pl.kernel(
        _sc_mask_body,
        out_type=jax.ShapeDtypeStruct((NC, N_EDGES), jnp.float32),
        mesh=mesh,
        scratch_types=[
            pltpu.VMEM((EPT,), jnp.int32),
            pltpu.VMEM((EPT,), jnp.int32),
            pltpu.VMEM((NCHUNK, CHUNK), jnp.int32),
            pltpu.VMEM((CHUNK,), jnp.float32),
            pltpu.VMEM((ZB,), jnp.float32),
            pltpu.VMEM_SHARED((N_EDGES,), jnp.float32),
            pltpu.SemaphoreType.DMA,
        ],
    )(edge_index)


def _prep_layer(x, wa, ba_ref, wb_ref, d_in, a_s, b_s, wblk_s):
    wtop = wa[:d_in]
    wbot = wa[d_in:]
    a_s[...] = (jnp.dot(x, wtop - wbot, preferred_element_type=jnp.float32)
                + ba_ref[...])
    b_s[...] = jnp.dot(x, wbot, preferred_element_type=jnp.float32)
    wb = wb_ref[...]
    wt = jnp.tile(wb, (4, 4))
    ri = lax.broadcasted_iota(jnp.int32, (4 * H, 4 * H), 0)
    ci = lax.broadcasted_iota(jnp.int32, (4 * H, 4 * H), 1)
    wblk_s[...] = jnp.where((ri // H) == (ci // H), wt, 0.0)


def _layer_step(i_loc, bb_ref, a_s, b_s, wblk_s, pent_s, h_s):
    a_blk = a_s[pl.ds(i_loc * BD, BD), :]
    a2d = jnp.concatenate(
        [a_blk[k * NG:(k + 1) * NG, :] for k in range(4)], axis=1)
    b4 = jnp.tile(b_s[...], (1, 4))
    lhs = jnp.maximum(
        jnp.broadcast_to(b4[None], (NG, N_NODES, 4 * H)).reshape(
            NG * N_NODES, 4 * H)
        + jnp.broadcast_to(a2d[:, None, :], (NG, N_NODES, 4 * H)).reshape(
            NG * N_NODES, 4 * H),
        0.0)
    r = jnp.dot(lhs, wblk_s[...], preferred_element_type=jnp.float32)
    pent_blk = pent_s[pl.ds(i_loc * BD, BD), :].T
    pieces = []
    for g in range(NG):
        cols = [jnp.broadcast_to(pent_blk[:, k * NG + g][:, None],
                                 (N_NODES, H)) for k in range(4)]
        pieces.append(jnp.concatenate(cols, axis=1))
    pen_all = jnp.concatenate(pieces, axis=0)
    m = (r + pen_all).reshape(NG, N_NODES, 4 * H).max(axis=1)
    hq = jnp.maximum(m + jnp.tile(bb_ref[...], (1, 4)), 0.0)
    for k in range(4):
        h_s[pl.ds(i_loc * BD + k * NG, NG), :] = hq[:, k * H:(k + 1) * H]


def _fused_body(x_ref, w1_ref, b1_ref, w2_ref, b2_ref, w3_ref, b3_ref,
                w4_ref, b4_ref, wl_ref, bl_ref, wo_ref, bo_ref, cnt_ref,
                out_ref, a_s, b_s, wblk_s, pent_s, h1_s, h2_s):
    i = pl.program_id(0)

    @pl.when(i == 0)
    def _():
        cnt = cnt_ref[...]
        pent_s[...] = jnp.where((cnt[0] + cnt[1]) > 0.0, 0.0,
                                -1e30)
        _prep_layer(x_ref[...], w1_ref[...], b1_ref, w2_ref,
                    x_ref.shape[1], a_s, b_s, wblk_s)

    @pl.when(i < 8)
    def _():
        _layer_step(i, b2_ref, a_s, b_s, wblk_s, pent_s, h1_s)

    @pl.when(i == 8)
    def _():
        _prep_layer(h1_s[...], w3_ref[...], b3_ref, w4_ref, H, a_s, b_s,
                    wblk_s)

    @pl.when(jnp.logical_and(i >= 8, i < 16))
    def _():
        _layer_step(i - 8, b4_ref, a_s, b_s, wblk_s, pent_s, h2_s)

    @pl.when(i == 16)
    def _():
        ht = h2_s[...].T
        acc = bl_ref[...]
        for h in range(H):
            acc = acc + jnp.dot(ht[h][None], wl_ref[h],
                                preferred_element_type=jnp.float32)
        z1 = jnp.maximum(acc, 0.0)
        z2 = jnp.maximum(
            jnp.dot(z1, wo_ref[...], preferred_element_type=jnp.float32)
            + bo_ref[...], 0.0)
        mx = jnp.max(z2, axis=1, keepdims=True)
        e = jnp.exp(z2 - mx)
        out_ref[...] = e / jnp.sum(e, axis=1, keepdims=True)


def _fused(x, w1, b1, w2, b2, w3, b3, w4, b4, wl, bl, wo, bo, cnt3):
    full = lambda shape: pl.BlockSpec(shape, lambda i: tuple(0 for _ in shape))
    return pl.pallas_call(
        _fused_body,
        grid=(17,),
        in_specs=[
            full((N_NODES, DENSE)),
            full((2 * DENSE, H)), full((1, H)),
            full((H, H)), full((1, H)),
            full((2 * H, H)), full((1, H)),
            full((H, H)), full((1, H)),
            full((H, N_NODES, DENSE)), full((1, DENSE)),
            full((DENSE, N_NODES)), full((1, N_NODES)),
            full((NC, N_NODES, N_NODES)),
        ],
        out_specs=full((1, N_NODES)),
        out_shape=jax.ShapeDtypeStruct((1, N_NODES), jnp.float32),
        scratch_shapes=[
            pltpu.VMEM((N_NODES, H), jnp.float32),
            pltpu.VMEM((N_NODES, H), jnp.float32),
            pltpu.VMEM((4 * H, 4 * H), jnp.float32),
            pltpu.VMEM((N_NODES, N_NODES), jnp.float32),
            pltpu.VMEM((N_NODES, H), jnp.float32),
            pltpu.VMEM((N_NODES, H), jnp.float32),
        ],
    )(x, w1, b1.reshape(1, H), w2, b2.reshape(1, H),
      w3, b3.reshape(1, H), w4, b4.reshape(1, H),
      wl.reshape(N_NODES, H, DENSE).transpose(1, 0, 2),
      bl.reshape(1, DENSE), wo, bo.reshape(1, N_NODES), cnt3)


def kernel(x, edge_index, W1, b1, W2, b2, W3, b3, W4, b4, Wl, bl, Wo, bo):
    cnt = _sc_mask(edge_index)
    cnt3 = cnt.reshape(NC, N_NODES, N_NODES)
    out = _fused(x, W1, b1, W2, b2, W3, b3, W4, b4, Wl, bl, Wo, bo, cnt3)
    return out.reshape(N_NODES)


# speedup vs baseline: 1.3170x; 1.3170x over previous
import jax
import jax.numpy as jnp
from jax import lax
from jax.experimental import pallas as pl
from jax.experimental.pallas import tpu as pltpu
from jax.experimental.pallas import tpu_sc as plsc

N_NODES = 512
N_EDGES = 262144
H = 32
DENSE = 128
NC = 2
NS = 16
NW = NC * NS
EPT = N_EDGES // NW
CHUNK = 128
NCHUNK = EPT // CHUNK
SEG = N_EDGES // NS
ZB = 2048
BD = 64
NG = BD // 4


def _sc_mask_body(e_hbm, out_hbm, src_v, dst_v, idx_v, ones_v, zeros_v,
                  mask_sh, sem):
    cid = lax.axis_index("c")
    sid = lax.axis_index("s")
    wid = cid * NS + sid
    base = wid * EPT

    pltpu.sync_copy(e_hbm.at[0, pl.ds(base, EPT)], src_v)
    pltpu.sync_copy(e_hbm.at[1, pl.ds(base, EPT)], dst_v)

    for k in range(CHUNK // 16):
        ones_v[pl.ds(k * 16, 16)] = jnp.ones((16,), jnp.float32)

    def _fill_zeros(i, c):
        for k in range(8):
            zeros_v[pl.ds((i * 8 + k) * 16, 16)] = jnp.zeros((16,),
                                                             jnp.float32)
        return c

    lax.fori_loop(0, ZB // 128, _fill_zeros, 0)

    for k in range(SEG // ZB):
        pltpu.sync_copy(zeros_v, mask_sh.at[pl.ds(sid * SEG + k * ZB, ZB)])

    def _codes(j, c):
        for k in range(8):
            off = j * CHUNK + k * 16
            s = src_v[pl.ds(off, 16)]
            d = dst_v[pl.ds(off, 16)]
            idx_v[j, pl.ds(k * 16, 16)] = d * N_NODES + s
        return c

    lax.fori_loop(0, NCHUNK, _codes, 0)
    plsc.subcore_barrier()

    def _scatter(g, c):
        sds = [pltpu.async_copy(ones_v, mask_sh.at[idx_v.at[g * 8 + k]], sem,
                                add=True)
               for k in range(8)]
        for sd in sds:
            sd.wait()
        return c

    lax.fori_loop(0, NCHUNK // 8, _scatter, 0)
    plsc.subcore_barrier()
    pltpu.sync_copy(mask_sh.at[pl.ds(sid * SEG, SEG)],
                    out_hbm.at[cid, pl.ds(sid * SEG, SEG)])


def _sc_mask(edge_index):
    mesh = plsc.VectorSubcoreMesh(core_axis_name="c", subcore_axis_name="s")
    return pl.kernel(
        _sc_mask_body,
        out_type=jax.ShapeDtypeStruct((NC, N_EDGES), jnp.float32),
        mesh=mesh,
        scratch_types=[
            pltpu.VMEM((EPT,), jnp.int32),
            pltpu.VMEM((EPT,), jnp.int32),
            pltpu.VMEM((NCHUNK, CHUNK), jnp.int32),
            pltpu.VMEM((CHUNK,), jnp.float32),
            pltpu.VMEM((ZB,), jnp.float32),
            pltpu.VMEM_SHARED((N_EDGES,), jnp.float32),
            pltpu.SemaphoreType.DMA,
        ],
    )(edge_index)


def _prep_layer(x, wa, ba_ref, wb_ref, d_in, a_s, bt_s, wbe_s):
    wtop = wa[:d_in]
    wbot = wa[d_in:]
    a_s[...] = (jnp.dot(x, wtop - wbot, preferred_element_type=jnp.float32)
                + ba_ref[...])
    bt_s[...] = jnp.dot(x, wbot, preferred_element_type=jnp.float32).T
    wbe_s[...] = jnp.concatenate(
        [wb_ref[...].T, jnp.ones((H, 1), jnp.float32)], axis=1)


def _layer_step(i_loc, bb_ref, cnt_ref, a_s, bt_s, wbe_s, h_s):
    a_t = a_s[pl.ds(i_loc * BD, BD), :].T
    pre = jnp.maximum(a_t[:, :, None] + bt_s[...][:, None, :], 0.0)
    pre2 = pre.reshape(H, BD * N_NODES)
    pen = jnp.where(
        (cnt_ref[0, pl.ds(i_loc * BD, BD), :]
         + cnt_ref[1, pl.ds(i_loc * BD, BD), :]) > 0.0,
        0.0, -1e30).reshape(1, BD * N_NODES)
    rhs = jnp.concatenate([pre2, pen], axis=0)
    r2 = jnp.dot(wbe_s[...], rhs, preferred_element_type=jnp.float32)
    m = r2.reshape(H, BD, N_NODES).max(axis=2)
    h_s[pl.ds(i_loc * BD, BD), :] = jnp.maximum(m.T + bb_ref[...], 0.0)


def _fused_body(x_ref, w1_ref, b1_ref, w2_ref, b2_ref, w3_ref, b3_ref,
                w4_ref, b4_ref, wl_ref, bl_ref, wo_ref, bo_ref, cnt_ref,
                out_ref, a_s, bt_s, wbe_s, h1_s, h2_s):
    i = pl.program_id(0)

    @pl.when(i == 0)
    def _():
        _prep_layer(x_ref[...], w1_ref[...], b1_ref, w2_ref,
                    x_ref.shape[1], a_s, bt_s, wbe_s)

    @pl.when(i < 8)
    def _():
        _layer_step(i, b2_ref, cnt_ref, a_s, bt_s, wbe_s, h1_s)

    @pl.when(i == 8)
    def _():
        _prep_layer(h1_s[...], w3_ref[...], b3_ref, w4_ref, H, a_s, bt_s,
                    wbe_s)

    @pl.when(jnp.logical_and(i >= 8, i < 16))
    def _():
        _layer_step(i - 8, b4_ref, cnt_ref, a_s, bt_s, wbe_s, h2_s)

    @pl.when(i == 16)
    def _():
        ht = h2_s[...].T
        acc = bl_ref[...]
        for h in range(H):
            acc = acc + jnp.dot(ht[h][None], wl_ref[h],
                                preferred_element_type=jnp.float32)
        z1 = jnp.maximum(acc, 0.0)
        z2 = jnp.maximum(
            jnp.dot(z1, wo_ref[...], preferred_element_type=jnp.float32)
            + bo_ref[...], 0.0)
        mx = jnp.max(z2, axis=1, keepdims=True)
        e = jnp.exp(z2 - mx)
        out_ref[...] = e / jnp.sum(e, axis=1, keepdims=True)


def _fused(x, w1, b1, w2, b2, w3, b3, w4, b4, wl, bl, wo, bo, cnt3):
    full = lambda shape: pl.BlockSpec(shape, lambda i: tuple(0 for _ in shape))
    return pl.pallas_call(
        _fused_body,
        grid=(17,),
        in_specs=[
            full((N_NODES, DENSE)),
            full((2 * DENSE, H)), full((1, H)),
            full((H, H)), full((1, H)),
            full((2 * H, H)), full((1, H)),
            full((H, H)), full((1, H)),
            full((H, N_NODES, DENSE)), full((1, DENSE)),
            full((DENSE, N_NODES)), full((1, N_NODES)),
            full((NC, N_NODES, N_NODES)),
        ],
        out_specs=full((1, N_NODES)),
        out_shape=jax.ShapeDtypeStruct((1, N_NODES), jnp.float32),
        scratch_shapes=[
            pltpu.VMEM((N_NODES, H), jnp.float32),
            pltpu.VMEM((H, N_NODES), jnp.float32),
            pltpu.VMEM((H, H + 1), jnp.float32),
            pltpu.VMEM((N_NODES, H), jnp.float32),
            pltpu.VMEM((N_NODES, H), jnp.float32),
        ],
    )(x, w1, b1.reshape(1, H), w2, b2.reshape(1, H),
      w3, b3.reshape(1, H), w4, b4.reshape(1, H),
      wl.reshape(N_NODES, H, DENSE).transpose(1, 0, 2),
      bl.reshape(1, DENSE), wo, bo.reshape(1, N_NODES), cnt3)


def kernel(x, edge_index, W1, b1, W2, b2, W3, b3, W4, b4, Wl, bl, Wo, bo):
    cnt = _sc_mask(edge_index)
    cnt3 = cnt.reshape(NC, N_NODES, N_NODES)
    out = _fused(x, W1, b1, W2, b2, W3, b3, W4, b4, Wl, bl, Wo, bo, cnt3)
    return out.reshape(N_NODES)


# speedup vs baseline: 1.3613x; 1.0336x over previous
import jax
import jax.numpy as jnp
from jax import lax
from jax.experimental import pallas as pl
from jax.experimental.pallas import tpu as pltpu
from jax.experimental.pallas import tpu_sc as plsc

N_NODES = 512
N_EDGES = 262144
H = 32
DENSE = 128
NC = 2
NS = 16
NW = NC * NS
EPT = N_EDGES // NW
CHUNK = 128
NCHUNK = EPT // CHUNK
SEG = N_EDGES // NS
ZB = 2048
BD = 128
NBL = N_NODES // BD


def _sc_mask_body(e_hbm, out_hbm, src_v, dst_v, idx_v, ones_v, zeros_v,
                  mask_sh, sem):
    cid = lax.axis_index("c")
    sid = lax.axis_index("s")
    wid = cid * NS + sid
    base = wid * EPT

    pltpu.sync_copy(e_hbm.at[0, pl.ds(base, EPT)], src_v)
    pltpu.sync_copy(e_hbm.at[1, pl.ds(base, EPT)], dst_v)

    for k in range(CHUNK // 16):
        ones_v[pl.ds(k * 16, 16)] = jnp.ones((16,), jnp.float32)

    def _fill_zeros(i, c):
        for k in range(8):
            zeros_v[pl.ds((i * 8 + k) * 16, 16)] = jnp.zeros((16,),
                                                             jnp.float32)
        return c

    lax.fori_loop(0, ZB // 128, _fill_zeros, 0)

    for k in range(SEG // ZB):
        pltpu.sync_copy(zeros_v, mask_sh.at[pl.ds(sid * SEG + k * ZB, ZB)])

    def _codes(j, c):
        for k in range(8):
            off = j * CHUNK + k * 16
            s = src_v[pl.ds(off, 16)]
            d = dst_v[pl.ds(off, 16)]
            idx_v[j, pl.ds(k * 16, 16)] = d * N_NODES + s
        return c

    lax.fori_loop(0, NCHUNK, _codes, 0)
    plsc.subcore_barrier()

    def _scatter(g, c):
        sds = [pltpu.async_copy(ones_v, mask_sh.at[idx_v.at[g * 8 + k]], sem,
                                add=True)
               for k in range(8)]
        for sd in sds:
            sd.wait()
        return c

    lax.fori_loop(0, NCHUNK // 8, _scatter, 0)
    plsc.subcore_barrier()
    pltpu.sync_copy(mask_sh.at[pl.ds(sid * SEG, SEG)],
                    out_hbm.at[cid, pl.ds(sid * SEG, SEG)])


def _sc_mask(edge_index):
    mesh = plsc.VectorSubcoreMesh(core_axis_name="c", subcore_axis_name="s")
    return pl.kernel(
        _sc_mask_body,
        out_type=jax.ShapeDtypeStruct((NC, N_EDGES), jnp.float32),
        mesh=mesh,
        scratch_types=[
            pltpu.VMEM((EPT,), jnp.int32),
            pltpu.VMEM((EPT,), jnp.int32),
            pltpu.VMEM((NCHUNK, CHUNK), jnp.int32),
            pltpu.VMEM((CHUNK,), jnp.float32),
            pltpu.VMEM((ZB,), jnp.float32),
            pltpu.VMEM_SHARED((N_EDGES,), jnp.float32),
            pltpu.SemaphoreType.DMA,
        ],
    )(edge_index)


def _prep_layer(xt, wat, bat_ref, wbt_ref, d_in, at_s, bt_s, wbe_s):
    wdt = wat[:, :d_in] - wat[:, d_in:]
    at_s[...] = (jnp.dot(wdt, xt, preferred_element_type=jnp.float32)
                 + bat_ref[...])
    bt_s[...] = jnp.dot(wat[:, d_in:], xt,
                        preferred_element_type=jnp.float32)
    wbe_s[...] = jnp.concatenate(
        [wbt_ref[...], jnp.ones((H, 1), jnp.float32)], axis=1)


def _layer_step(i_loc, bbt_ref, cnt_ref, at_s, bt_s, wbe_s, ht_s):
    a_t = at_s[:, pl.ds(i_loc * BD, BD)]
    pre = jnp.maximum(a_t[:, :, None] + bt_s[...][:, None, :], 0.0)
    pre2 = pre.reshape(H, BD * N_NODES)
    pen = jnp.where(
        (cnt_ref[0, pl.ds(i_loc, 1), :] + cnt_ref[1, pl.ds(i_loc, 1), :])
        > 0.0, 0.0, -1e30)
    rhs = jnp.concatenate([pre2, pen], axis=0)
    r2 = jnp.dot(wbe_s[...], rhs, preferred_element_type=jnp.float32)
    m = r2.reshape(H, BD, N_NODES).max(axis=2)
    ht_s[:, pl.ds(i_loc * BD, BD)] = jnp.maximum(m + bbt_ref[...], 0.0)


def _fused_body(xt_ref, w1_ref, b1_ref, w2_ref, b2_ref, w3_ref, b3_ref,
                w4_ref, b4_ref, wl_ref, bl_ref, wo_ref, bo_ref, cnt_ref,
                out_ref, at_s, bt_s, wbe_s, h1_s, h2_s):
    i = pl.program_id(0)

    @pl.when(i == 0)
    def _():
        _prep_layer(xt_ref[...], w1_ref[...], b1_ref, w2_ref,
                    xt_ref.shape[0], at_s, bt_s, wbe_s)

    @pl.when(i < NBL)
    def _():
        _layer_step(i, b2_ref, cnt_ref, at_s, bt_s, wbe_s, h1_s)

    @pl.when(i == NBL)
    def _():
        _prep_layer(h1_s[...], w3_ref[...], b3_ref, w4_ref, H, at_s, bt_s,
                    wbe_s)

    @pl.when(jnp.logical_and(i >= NBL, i < 2 * NBL))
    def _():
        _layer_step(i - NBL, b4_ref, cnt_ref, at_s, bt_s, wbe_s, h2_s)

    @pl.when(i == 2 * NBL)
    def _():
        ht = h2_s[...]
        acc = bl_ref[...]
        for h in range(H):
            acc = acc + jnp.dot(ht[h][None], wl_ref[h],
                                preferred_element_type=jnp.float32)
        z1 = jnp.maximum(acc, 0.0)
        z2 = jnp.maximum(
            jnp.dot(z1, wo_ref[...], preferred_element_type=jnp.float32)
            + bo_ref[...], 0.0)
        mx = jnp.max(z2, axis=1, keepdims=True)
        e = jnp.exp(z2 - mx)
        out_ref[...] = e / jnp.sum(e, axis=1, keepdims=True)


def _fused(x, w1, b1, w2, b2, w3, b3, w4, b4, wl, bl, wo, bo, cnt):
    full = lambda shape: pl.BlockSpec(shape, lambda i: tuple(0 for _ in shape))
    return pl.pallas_call(
        _fused_body,
        grid=(2 * NBL + 1,),
        in_specs=[
            full((DENSE, N_NODES)),
            full((H, 2 * DENSE)), full((H, 1)),
            full((H, H)), full((H, 1)),
            full((H, 2 * H)), full((H, 1)),
            full((H, H)), full((H, 1)),
            full((H, N_NODES, DENSE)), full((1, DENSE)),
            full((DENSE, N_NODES)), full((1, N_NODES)),
            full((NC, N_NODES // BD, BD * N_NODES)),
        ],
        out_specs=full((1, N_NODES)),
        out_shape=jax.ShapeDtypeStruct((1, N_NODES), jnp.float32),
        scratch_shapes=[
            pltpu.VMEM((H, N_NODES), jnp.float32),
            pltpu.VMEM((H, N_NODES), jnp.float32),
            pltpu.VMEM((H, H + 1), jnp.float32),
            pltpu.VMEM((H, N_NODES), jnp.float32),
            pltpu.VMEM((H, N_NODES), jnp.float32),
        ],
    )(x.T, w1.T, b1.reshape(H, 1), w2.T, b2.reshape(H, 1),
      w3.T, b3.reshape(H, 1), w4.T, b4.reshape(H, 1),
      wl.reshape(N_NODES, H, DENSE).transpose(1, 0, 2),
      bl.reshape(1, DENSE), wo, bo.reshape(1, N_NODES),
      cnt.reshape(NC, N_NODES // BD, BD * N_NODES))


def kernel(x, edge_index, W1, b1, W2, b2, W3, b3, W4, b4, Wl, bl, Wo, bo):
    cnt = _sc_mask(edge_index)
    out = _fused(x, W1, b1, W2, b2, W3, b3, W4, b4, Wl, bl, Wo, bo, cnt)
    return out.reshape(N_NODES)


# speedup vs baseline: 2.0227x; 1.4859x over previous
import jax
import jax.numpy as jnp
from jax import lax
from jax.experimental import pallas as pl
from jax.experimental.pallas import tpu as pltpu
from jax.experimental.pallas import tpu_sc as plsc

N_NODES = 512
N_EDGES = 262144
H = 32
DENSE = 128
NC = 2
NS = 16
NW = NC * NS
EPT = N_EDGES // NW
CHUNK = 128
NCHUNK = EPT // CHUNK
SEG = N_EDGES // NS
ZB = 2048
BD = 128
NBL = N_NODES // BD


def _sc_mask_body(e_hbm, out_hbm, src_v, dst_v, idx_v, ones_v, zeros_v,
                  mask_sh, sem):
    cid = lax.axis_index("c")
    sid = lax.axis_index("s")
    wid = cid * NS + sid
    base = wid * EPT

    pltpu.sync_copy(e_hbm.at[0, pl.ds(base, EPT)], src_v)
    pltpu.sync_copy(e_hbm.at[1, pl.ds(base, EPT)], dst_v)

    for k in range(CHUNK // 16):
        ones_v[pl.ds(k * 16, 16)] = jnp.ones((16,), jnp.float32)

    def _fill_zeros(i, c):
        for k in range(8):
            zeros_v[pl.ds((i * 8 + k) * 16, 16)] = jnp.zeros((16,),
                                                             jnp.float32)
        return c

    lax.fori_loop(0, ZB // 128, _fill_zeros, 0)

    for k in range(SEG // ZB):
        pltpu.sync_copy(zeros_v, mask_sh.at[pl.ds(sid * SEG + k * ZB, ZB)])

    def _codes(j, c):
        for k in range(8):
            off = j * CHUNK + k * 16
            s = src_v[pl.ds(off, 16)]
            d = dst_v[pl.ds(off, 16)]
            dh = lax.shift_right_logical(d, 7)
            dl = jnp.bitwise_and(d, BD - 1)
            idx_v[j, pl.ds(k * 16, 16)] = (dh * (BD * N_NODES)
                                           + s * BD + dl)
        return c

    lax.fori_loop(0, NCHUNK, _codes, 0)
    plsc.subcore_barrier()

    def _scatter(g, c):
        sds = [pltpu.async_copy(ones_v, mask_sh.at[idx_v.at[g * 8 + k]], sem,
                                add=True)
               for k in range(8)]
        for sd in sds:
            sd.wait()
        return c

    lax.fori_loop(0, NCHUNK // 8, _scatter, 0)
    plsc.subcore_barrier()
    pltpu.sync_copy(mask_sh.at[pl.ds(sid * SEG, SEG)],
                    out_hbm.at[cid, pl.ds(sid * SEG, SEG)])


def _sc_mask(edge_index):
    mesh = plsc.VectorSubcoreMesh(core_axis_name="c", subcore_axis_name="s")
    return pl.kernel(
        _sc_mask_body,
        out_type=jax.ShapeDtypeStruct((NC, N_EDGES), jnp.float32),
        mesh=mesh,
        scratch_types=[
            pltpu.VMEM((EPT,), jnp.int32),
            pltpu.VMEM((EPT,), jnp.int32),
            pltpu.VMEM((NCHUNK, CHUNK), jnp.int32),
            pltpu.VMEM((CHUNK,), jnp.float32),
            pltpu.VMEM((ZB,), jnp.float32),
            pltpu.VMEM_SHARED((N_EDGES,), jnp.float32),
            pltpu.SemaphoreType.DMA,
        ],
    )(edge_index)


def _prep_layer(xt, wat, bat_ref, wbt_ref, d_in, at_s, btr_s, wbe_s):
    wdt = wat[:, :d_in] - wat[:, d_in:]
    at_s[...] = (jnp.dot(wdt, xt, preferred_element_type=jnp.float32)
                 + bat_ref[...])
    bt = jnp.dot(wat[:, d_in:], xt, preferred_element_type=jnp.float32)
    btr_s[...] = jnp.broadcast_to(bt[:, :, None],
                                  (H, N_NODES, BD)).reshape(H, N_NODES * BD)
    wbe_s[...] = jnp.concatenate(
        [wbt_ref[...], jnp.ones((H, 1), jnp.float32)], axis=1)


def _layer_step(i_loc, bbt_ref, cnt_ref, at_s, btr_s, wbe_s, ht_s):
    a_t = at_s[:, pl.ds(i_loc * BD, BD)]
    pre2 = jnp.maximum(jnp.tile(a_t, (1, N_NODES)) + btr_s[...],
                       0.0)
    pen = jnp.where(
        (cnt_ref[0, pl.ds(i_loc, 1), :] + cnt_ref[1, pl.ds(i_loc, 1), :])
        > 0.0, 0.0, -1e30)
    rhs = jnp.concatenate([pre2, pen], axis=0)
    r2 = jnp.dot(wbe_s[...], rhs, preferred_element_type=jnp.float32)
    m = r2
    for _ in range(9):
        half = m.shape[1] // 2
        m = jnp.maximum(m[:, :half], m[:, half:])
    ht_s[:, pl.ds(i_loc * BD, BD)] = jnp.maximum(m + bbt_ref[...], 0.0)


def _fused_body(xt_ref, w1_ref, b1_ref, w2_ref, b2_ref, w3_ref, b3_ref,
                w4_ref, b4_ref, wl_ref, bl_ref, wo_ref, bo_ref, cnt_ref,
                out_ref, at_s, btr_s, wbe_s, h1_s, h2_s):
    i = pl.program_id(0)

    @pl.when(i == 0)
    def _():
        _prep_layer(xt_ref[...], w1_ref[...], b1_ref, w2_ref,
                    xt_ref.shape[0], at_s, btr_s, wbe_s)

    @pl.when(i < NBL)
    def _():
        _layer_step(i, b2_ref, cnt_ref, at_s, btr_s, wbe_s, h1_s)

    @pl.when(i == NBL)
    def _():
        _prep_layer(h1_s[...], w3_ref[...], b3_ref, w4_ref, H, at_s, btr_s,
                    wbe_s)

    @pl.when(jnp.logical_and(i >= NBL, i < 2 * NBL))
    def _():
        _layer_step(i - NBL, b4_ref, cnt_ref, at_s, btr_s, wbe_s, h2_s)

    @pl.when(i == 2 * NBL)
    def _():
        ht = h2_s[...]
        acc = bl_ref[...]
        for h in range(H):
            acc = acc + jnp.dot(ht[h][None], wl_ref[h],
                                preferred_element_type=jnp.float32)
        z1 = jnp.maximum(acc, 0.0)
        z2 = jnp.maximum(
            jnp.dot(z1, wo_ref[...], preferred_element_type=jnp.float32)
            + bo_ref[...], 0.0)
        mx = jnp.max(z2, axis=1, keepdims=True)
        e = jnp.exp(z2 - mx)
        out_ref[...] = e / jnp.sum(e, axis=1, keepdims=True)


def _fused(x, w1, b1, w2, b2, w3, b3, w4, b4, wl, bl, wo, bo, cnt):
    full = lambda shape: pl.BlockSpec(shape, lambda i: tuple(0 for _ in shape))
    return pl.pallas_call(
        _fused_body,
        grid=(2 * NBL + 1,),
        in_specs=[
            full((DENSE, N_NODES)),
            full((H, 2 * DENSE)), full((H, 1)),
            full((H, H)), full((H, 1)),
            full((H, 2 * H)), full((H, 1)),
            full((H, H)), full((H, 1)),
            full((H, N_NODES, DENSE)), full((1, DENSE)),
            full((DENSE, N_NODES)), full((1, N_NODES)),
            full((NC, N_NODES // BD, BD * N_NODES)),
        ],
        out_specs=full((1, N_NODES)),
        out_shape=jax.ShapeDtypeStruct((1, N_NODES), jnp.float32),
        scratch_shapes=[
            pltpu.VMEM((H, N_NODES), jnp.float32),
            pltpu.VMEM((H, N_NODES * BD), jnp.float32),
            pltpu.VMEM((H, H + 1), jnp.float32),
            pltpu.VMEM((H, N_NODES), jnp.float32),
            pltpu.VMEM((H, N_NODES), jnp.float32),
        ],
    )(x.T, w1.T, b1.reshape(H, 1), w2.T, b2.reshape(H, 1),
      w3.T, b3.reshape(H, 1), w4.T, b4.reshape(H, 1),
      wl.reshape(N_NODES, H, DENSE).transpose(1, 0, 2),
      bl.reshape(1, DENSE), wo, bo.reshape(1, N_NODES),
      cnt.reshape(NC, N_NODES // BD, BD * N_NODES))


def kernel(x, edge_index, W1, b1, W2, b2, W3, b3, W4, b4, Wl, bl, Wo, bo):
    cnt = _sc_mask(edge_index)
    out = _fused(x, W1, b1, W2, b2, W3, b3, W4, b4, Wl, bl, Wo, bo, cnt)
    return out.reshape(N_NODES)


# speedup vs baseline: 2.0784x; 1.0276x over previous
import jax
import jax.numpy as jnp
from jax import lax
from jax.experimental import pallas as pl
from jax.experimental.pallas import tpu as pltpu
from jax.experimental.pallas import tpu_sc as plsc

N_NODES = 512
N_EDGES = 262144
H = 32
DENSE = 128
NC = 2
NS = 16
NW = NC * NS
EPT = N_EDGES // NW
CHUNK = 128
NCHUNK = EPT // CHUNK
SEG = N_EDGES // NS
ZB = 2048
BD = 128
NBL = N_NODES // BD


def _sc_mask_body(e_hbm, out_hbm, src_v, dst_v, idx_v, ones_v, zeros_v,
                  mask_sh, sem):
    cid = lax.axis_index("c")
    sid = lax.axis_index("s")
    wid = cid * NS + sid
    base = wid * EPT

    pltpu.sync_copy(e_hbm.at[pl.ds(base, EPT)], src_v)
    pltpu.sync_copy(e_hbm.at[pl.ds(N_EDGES + base, EPT)], dst_v)

    for k in range(CHUNK // 16):
        ones_v[pl.ds(k * 16, 16)] = jnp.ones((16,), jnp.float32)

    def _fill_zeros(i, c):
        for k in range(8):
            zeros_v[pl.ds((i * 8 + k) * 16, 16)] = jnp.zeros((16,),
                                                             jnp.float32)
        return c

    lax.fori_loop(0, ZB // 128, _fill_zeros, 0)

    for k in range(SEG // ZB):
        pltpu.sync_copy(zeros_v, mask_sh.at[pl.ds(sid * SEG + k * ZB, ZB)])

    def _codes(j, c):
        for k in range(8):
            off = j * CHUNK + k * 16
            s = src_v[pl.ds(off, 16)]
            d = dst_v[pl.ds(off, 16)]
            dh = lax.shift_right_logical(d, 7)
            dl = jnp.bitwise_and(d, BD - 1)
            idx_v[j, pl.ds(k * 16, 16)] = (dh * (BD * N_NODES)
                                           + s * BD + dl)
        return c

    lax.fori_loop(0, NCHUNK, _codes, 0)
    plsc.subcore_barrier()

    def _scatter(g, c):
        sds = [pltpu.async_copy(ones_v, mask_sh.at[idx_v.at[g * 8 + k]], sem,
                                add=True)
               for k in range(8)]
        for sd in sds:
            sd.wait()
        return c

    lax.fori_loop(0, NCHUNK // 8, _scatter, 0)
    plsc.subcore_barrier()
    pltpu.sync_copy(mask_sh.at[pl.ds(sid * SEG, SEG)],
                    out_hbm.at[cid, pl.ds(sid * SEG, SEG)])


def _sc_mask(edge_index):
    mesh = plsc.VectorSubcoreMesh(core_axis_name="c", subcore_axis_name="s")
    return pl.kernel(
        _sc_mask_body,
        out_type=jax.ShapeDtypeStruct((NC, N_EDGES), jnp.float32),
        mesh=mesh,
        scratch_types=[
            pltpu.VMEM((EPT,), jnp.int32),
            pltpu.VMEM((EPT,), jnp.int32),
            pltpu.VMEM((NCHUNK, CHUNK), jnp.int32),
            pltpu.VMEM((CHUNK,), jnp.float32),
            pltpu.VMEM((ZB,), jnp.float32),
            pltpu.VMEM_SHARED((N_EDGES,), jnp.float32),
            pltpu.SemaphoreType.DMA,
        ],
    )(edge_index)


def _prep_layer(xt, wat, bat_ref, wbt_ref, d_in, at_s, btr_s, wbe_s):
    wdt = wat[:, :d_in] - wat[:, d_in:]
    at_s[...] = (jnp.dot(wdt, xt, preferred_element_type=jnp.float32)
                 + bat_ref[...])
    bt = jnp.dot(wat[:, d_in:], xt, preferred_element_type=jnp.float32)
    btr_s[...] = jnp.broadcast_to(bt[:, :, None],
                                  (H, N_NODES, BD)).reshape(H, N_NODES * BD)
    wbe_s[...] = jnp.concatenate(
        [wbt_ref[...], jnp.ones((H, 1), jnp.float32)], axis=1)


def _layer_step(i_loc, bbt_ref, cnt_ref, at_s, btr_s, wbe_s, ht_s):
    a_t = at_s[:, pl.ds(i_loc * BD, BD)]
    pre2 = jnp.maximum(jnp.tile(a_t, (1, N_NODES)) + btr_s[...],
                       0.0)
    pen = jnp.where(
        (cnt_ref[0, pl.ds(i_loc, 1), :] + cnt_ref[1, pl.ds(i_loc, 1), :])
        > 0.0, 0.0, -1e30)
    rhs = jnp.concatenate([pre2, pen], axis=0)
    r2 = jnp.dot(wbe_s[...], rhs, preferred_element_type=jnp.float32)
    m = r2
    for _ in range(9):
        half = m.shape[1] // 2
        m = jnp.maximum(m[:, :half], m[:, half:])
    ht_s[:, pl.ds(i_loc * BD, BD)] = jnp.maximum(m + bbt_ref[...], 0.0)


def _fused_body(xt_ref, w1_ref, b1_ref, w2_ref, b2_ref, w3_ref, b3_ref,
                w4_ref, b4_ref, wl_ref, bl_ref, wo_ref, bo_ref, cnt_ref,
                out_ref, at_s, btr_s, wbe_s, h1_s, h2_s, wlv_s, wl_sem):
    i = pl.program_id(0)

    @pl.when(i == 0)
    def _():
        pltpu.make_async_copy(wl_ref, wlv_s, wl_sem).start()
        _prep_layer(xt_ref[...], w1_ref[...], b1_ref, w2_ref,
                    xt_ref.shape[0], at_s, btr_s, wbe_s)

    @pl.when(i < NBL)
    def _():
        _layer_step(i, b2_ref, cnt_ref, at_s, btr_s, wbe_s, h1_s)

    @pl.when(i == NBL)
    def _():
        _prep_layer(h1_s[...], w3_ref[...], b3_ref, w4_ref, H, at_s, btr_s,
                    wbe_s)

    @pl.when(jnp.logical_and(i >= NBL, i < 2 * NBL))
    def _():
        _layer_step(i - NBL, b4_ref, cnt_ref, at_s, btr_s, wbe_s, h2_s)

    @pl.when(i == 2 * NBL)
    def _():
        pltpu.make_async_copy(wl_ref, wlv_s, wl_sem).wait()
        ht = h2_s[...]
        acc = bl_ref[...]
        for h in range(H):
            acc = acc + jnp.dot(ht[h][None], wlv_s[h],
                                preferred_element_type=jnp.float32)
        z1 = jnp.maximum(acc, 0.0)
        z2 = jnp.maximum(
            jnp.dot(z1, wo_ref[...], preferred_element_type=jnp.float32)
            + bo_ref[...], 0.0)
        mx = jnp.max(z2, axis=1, keepdims=True)
        e = jnp.exp(z2 - mx)
        out_ref[...] = e / jnp.sum(e, axis=1, keepdims=True)


def _fused(x, w1, b1, w2, b2, w3, b3, w4, b4, wl, bl, wo, bo, cnt):
    full = lambda shape: pl.BlockSpec(shape, lambda i: tuple(0 for _ in shape))
    return pl.pallas_call(
        _fused_body,
        grid=(2 * NBL + 1,),
        in_specs=[
            full((DENSE, N_NODES)),
            full((H, 2 * DENSE)), full((H, 1)),
            full((H, H)), full((H, 1)),
            full((H, 2 * H)), full((H, 1)),
            full((H, H)), full((H, 1)),
            pl.BlockSpec(memory_space=pl.ANY), full((1, DENSE)),
            full((DENSE, N_NODES)), full((1, N_NODES)),
            full((NC, N_NODES // BD, BD * N_NODES)),
        ],
        out_specs=full((1, N_NODES)),
        out_shape=jax.ShapeDtypeStruct((1, N_NODES), jnp.float32),
        scratch_shapes=[
            pltpu.VMEM((H, N_NODES), jnp.float32),
            pltpu.VMEM((H, N_NODES * BD), jnp.float32),
            pltpu.VMEM((H, H + 1), jnp.float32),
            pltpu.VMEM((H, N_NODES), jnp.float32),
            pltpu.VMEM((H, N_NODES), jnp.float32),
            pltpu.VMEM((H, N_NODES, DENSE), jnp.float32),
            pltpu.SemaphoreType.DMA,
        ],
    )(x.T, w1.T, b1.reshape(H, 1), w2.T, b2.reshape(H, 1),
      w3.T, b3.reshape(H, 1), w4.T, b4.reshape(H, 1),
      wl.reshape(N_NODES, H, DENSE).transpose(1, 0, 2),
      bl.reshape(1, DENSE), wo, bo.reshape(1, N_NODES),
      cnt.reshape(NC, N_NODES // BD, BD * N_NODES))


def kernel(x, edge_index, W1, b1, W2, b2, W3, b3, W4, b4, Wl, bl, Wo, bo):
    cnt = _sc_mask(edge_index.reshape(2 * N_EDGES))
    out = _fused(x, W1, b1, W2, b2, W3, b3, W4, b4, Wl, bl, Wo, bo, cnt)
    return out.reshape(N_NODES)


# speedup vs baseline: 2.0932x; 1.0071x over previous
import jax
import jax.numpy as jnp
from jax import lax
from jax.experimental import pallas as pl
from jax.experimental.pallas import tpu as pltpu
from jax.experimental.pallas import tpu_sc as plsc

N_NODES = 512
N_EDGES = 262144
H = 32
DENSE = 128
NC = 2
NS = 16
NW = NC * NS
EPT = N_EDGES // NW
CHUNK = 128
NCHUNK = EPT // CHUNK
SEG = N_EDGES // NS
ZB = 2048
BD = 128
NBL = N_NODES // BD


def _sc_mask_body(e_hbm, out_hbm, src_v, dst_v, idx_v, ones_v, zeros_v,
                  mask_sh, sem):
    cid = lax.axis_index("c")
    sid = lax.axis_index("s")
    wid = cid * NS + sid
    base = wid * EPT

    pltpu.sync_copy(e_hbm.at[pl.ds(base, EPT)], src_v)
    pltpu.sync_copy(e_hbm.at[pl.ds(N_EDGES + base, EPT)], dst_v)

    for k in range(CHUNK // 16):
        ones_v[pl.ds(k * 16, 16)] = jnp.ones((16,), jnp.float32)

    def _fill_zeros(i, c):
        for k in range(8):
            zeros_v[pl.ds((i * 8 + k) * 16, 16)] = jnp.zeros((16,),
                                                             jnp.float32)
        return c

    lax.fori_loop(0, ZB // 128, _fill_zeros, 0)

    for k in range(SEG // ZB):
        pltpu.sync_copy(zeros_v, mask_sh.at[pl.ds(sid * SEG + k * ZB, ZB)])

    def _codes(j, c):
        for k in range(8):
            off = j * CHUNK + k * 16
            s = src_v[pl.ds(off, 16)]
            d = dst_v[pl.ds(off, 16)]
            dh = lax.shift_right_logical(d, 7)
            dl = jnp.bitwise_and(d, BD - 1)
            idx_v[j, pl.ds(k * 16, 16)] = (dh * (BD * N_NODES)
                                           + s * BD + dl)
        return c

    lax.fori_loop(0, NCHUNK, _codes, 0)
    plsc.subcore_barrier()

    def _scatter(g, c):
        sds = [pltpu.async_copy(ones_v, mask_sh.at[idx_v.at[g * 8 + k]], sem,
                                add=True)
               for k in range(8)]
        for sd in sds:
            sd.wait()
        return c

    lax.fori_loop(0, NCHUNK // 8, _scatter, 0)
    plsc.subcore_barrier()
    pltpu.sync_copy(mask_sh.at[pl.ds(sid * SEG, SEG)],
                    out_hbm.at[cid, pl.ds(sid * SEG, SEG)])


def _sc_mask(edge_index):
    mesh = plsc.VectorSubcoreMesh(core_axis_name="c", subcore_axis_name="s")
    return pl.kernel(
        _sc_mask_body,
        out_type=jax.ShapeDtypeStruct((NC, N_EDGES), jnp.float32),
        mesh=mesh,
        scratch_types=[
            pltpu.VMEM((EPT,), jnp.int32),
            pltpu.VMEM((EPT,), jnp.int32),
            pltpu.VMEM((NCHUNK, CHUNK), jnp.int32),
            pltpu.VMEM((CHUNK,), jnp.float32),
            pltpu.VMEM((ZB,), jnp.float32),
            pltpu.VMEM_SHARED((N_EDGES,), jnp.float32),
            pltpu.SemaphoreType.DMA,
        ],
    )(edge_index)


def _prep_layer(xt, wat, bat_ref, wbt_ref, d_in, at_s, btr_s, wbe_s):
    wdt = wat[:, :d_in] - wat[:, d_in:]
    at_s[...] = (jnp.dot(wdt, xt, preferred_element_type=jnp.float32)
                 + bat_ref[...])
    bt = jnp.dot(wat[:, d_in:], xt, preferred_element_type=jnp.float32)
    btr_s[...] = jnp.broadcast_to(bt[:, :, None],
                                  (H, N_NODES, BD)).reshape(H, N_NODES * BD)
    wbe_s[...] = jnp.concatenate(
        [wbt_ref[...], jnp.ones((H, 1), jnp.float32)], axis=1)


def _layer_step(i_loc, bbt_ref, cnt_s, at_s, btr_s, wbe_s, ht_s):
    a_t = at_s[:, pl.ds(i_loc * BD, BD)]
    pre2 = jnp.maximum(jnp.tile(a_t, (1, N_NODES)) + btr_s[...],
                       0.0)
    blk = BD * N_NODES
    pen = jnp.where(
        (cnt_s[pl.ds(0, 1), pl.ds(i_loc * blk, blk)]
         + cnt_s[pl.ds(1, 1), pl.ds(i_loc * blk, blk)])
        > 0.0, 0.0, -1e30)
    rhs = jnp.concatenate([pre2, pen], axis=0)
    r2 = jnp.dot(wbe_s[...], rhs, preferred_element_type=jnp.float32)
    m = r2
    for _ in range(9):
        half = m.shape[1] // 2
        m = jnp.maximum(m[:, :half], m[:, half:])
    ht_s[:, pl.ds(i_loc * BD, BD)] = jnp.maximum(m + bbt_ref[...], 0.0)


def _fused_body(xt_ref, w1_ref, b1_ref, w2_ref, b2_ref, w3_ref, b3_ref,
                w4_ref, b4_ref, wl_ref, bl_ref, wo_ref, bo_ref, cnt_ref,
                out_ref, at_s, btr_s, wbe_s, h1_s, h2_s, wlv_s, wl_sem):
    i = pl.program_id(0)

    @pl.when(i == 0)
    def _():
        pltpu.make_async_copy(wl_ref, wlv_s, wl_sem).start()
        _prep_layer(xt_ref[...], w1_ref[...], b1_ref, w2_ref,
                    xt_ref.shape[0], at_s, btr_s, wbe_s)

    @pl.when(i < NBL)
    def _():
        _layer_step(i, b2_ref, cnt_ref, at_s, btr_s, wbe_s, h1_s)

    @pl.when(i == NBL)
    def _():
        _prep_layer(h1_s[...], w3_ref[...], b3_ref, w4_ref, H, at_s, btr_s,
                    wbe_s)

    @pl.when(jnp.logical_and(i >= NBL, i < 2 * NBL))
    def _():
        _layer_step(i - NBL, b4_ref, cnt_ref, at_s, btr_s, wbe_s, h2_s)

    @pl.when(i == 2 * NBL)
    def _():
        pltpu.make_async_copy(wl_ref, wlv_s, wl_sem).wait()
        ht = h2_s[...]
        acc = bl_ref[...]
        for h in range(H):
            acc = acc + jnp.dot(ht[h][None], wlv_s[h],
                                preferred_element_type=jnp.float32)
        z1 = jnp.maximum(acc, 0.0)
        z2 = jnp.maximum(
            jnp.dot(z1, wo_ref[...], preferred_element_type=jnp.float32)
            + bo_ref[...], 0.0)
        mx = jnp.max(z2, axis=1, keepdims=True)
        e = jnp.exp(z2 - mx)
        out_ref[...] = e / jnp.sum(e, axis=1, keepdims=True)


def _fused(x, w1, b1, w2, b2, w3, b3, w4, b4, wl, bl, wo, bo, cnt):
    full = lambda shape: pl.BlockSpec(shape, lambda i: tuple(0 for _ in shape))
    return pl.pallas_call(
        _fused_body,
        grid=(2 * NBL + 1,),
        in_specs=[
            full((DENSE, N_NODES)),
            full((H, 2 * DENSE)), full((H, 1)),
            full((H, H)), full((H, 1)),
            full((H, 2 * H)), full((H, 1)),
            full((H, H)), full((H, 1)),
            pl.BlockSpec(memory_space=pl.ANY), full((1, DENSE)),
            full((DENSE, N_NODES)), full((1, N_NODES)),
            full((NC, N_EDGES)),
        ],
        out_specs=full((1, N_NODES)),
        out_shape=jax.ShapeDtypeStruct((1, N_NODES), jnp.float32),
        scratch_shapes=[
            pltpu.VMEM((H, N_NODES), jnp.float32),
            pltpu.VMEM((H, N_NODES * BD), jnp.float32),
            pltpu.VMEM((H, H + 1), jnp.float32),
            pltpu.VMEM((H, N_NODES), jnp.float32),
            pltpu.VMEM((H, N_NODES), jnp.float32),
            pltpu.VMEM((H, N_NODES, DENSE), jnp.float32),
            pltpu.SemaphoreType.DMA,
        ],
    )(x.T, w1.T, b1.reshape(H, 1), w2.T, b2.reshape(H, 1),
      w3.T, b3.reshape(H, 1), w4.T, b4.reshape(H, 1),
      wl.reshape(N_NODES, H, DENSE).transpose(1, 0, 2),
      bl.reshape(1, DENSE), wo, bo.reshape(1, N_NODES), cnt)


def kernel(x, edge_index, W1, b1, W2, b2, W3, b3, W4, b4, Wl, bl, Wo, bo):
    cnt = _sc_mask(edge_index.reshape(2 * N_EDGES))
    out = _fused(x, W1, b1, W2, b2, W3, b3, W4, b4, Wl, bl, Wo, bo, cnt)
    return out.reshape(N_NODES)
